# baseline (speedup 1.0000x reference)
import jax
import jax.numpy as jnp
from jax import lax
from jax.experimental import pallas as pl
from jax.experimental.pallas import tpu as pltpu
from jax.experimental.pallas import tpu_sc as plsc

N = 10000
E = 320000
NC = 2
NS = 16
NW = NC * NS
CH = 125
K = 80
R = 10240
RPT = R // NS
IZ = 128


def _wid():
    return lax.axis_index("c") * NS + lax.axis_index("s")


def _sc_mesh():
    return plsc.VectorSubcoreMesh(core_axis_name="c", subcore_axis_name="s")


def _sc_degree(dst3, const16):
    def body(dst_hbm, const_hbm, out_hbm, acc, dst_v, ones_v):
        c = lax.axis_index("c")
        s = lax.axis_index("s")
        pltpu.sync_copy(dst_hbm.at[_wid()], dst_v)
        pltpu.sync_copy(const_hbm.at[pl.ds(0, CH)], ones_v)
        base = s * RPT
        for t in range(RPT // IZ):
            pltpu.sync_copy(const_hbm.at[pl.ds(CH, IZ)],
                            acc.at[pl.ds(base + t * IZ, IZ)])
        plsc.subcore_barrier()

        def step(j, carry):
            pltpu.sync_copy(ones_v, acc.at[dst_v.at[j]], add=True)
            return carry

        lax.fori_loop(0, K, step, 0)
        plsc.subcore_barrier()
        pltpu.sync_copy(acc.at[pl.ds(base, RPT)],
                        out_hbm.at[c].at[pl.ds(base, RPT)])

    return pl.kernel(
        body,
        out_type=jax.ShapeDtypeStruct((NC, R, 16), jnp.float32),
        mesh=_sc_mesh(),
        compiler_params=pltpu.CompilerParams(use_tc_tiling_on_sc=False),
        scratch_types=[
            pltpu.VMEM_SHARED((R, 16), jnp.float32),
            pltpu.VMEM((K, CH), jnp.int32),
            pltpu.VMEM((CH, 16), jnp.float32),
        ],
    )(dst3, const16)


def _sc_scatter64(u_tab, src3, dst3, zeros64):
    def body(u_hbm, src_hbm, dst_hbm, z_hbm, out_hbm,
             acc, u_s, src_v, dst_v, rows, gsem, ssem):
        c = lax.axis_index("c")
        s = lax.axis_index("s")
        w = _wid()
        pltpu.sync_copy(src_hbm.at[w], src_v)
        pltpu.sync_copy(dst_hbm.at[w], dst_v)
        pltpu.sync_copy(u_hbm.at[pl.ds(s * (N // NS), N // NS)],
                        u_s.at[pl.ds(s * (N // NS), N // NS)])
        base = s * RPT
        for t in range(RPT // IZ):
            pltpu.sync_copy(z_hbm, acc.at[pl.ds(base + t * IZ, IZ)])
        plsc.subcore_barrier()

        pltpu.async_copy(u_s.at[src_v.at[0]], rows[0], gsem[0])

        def halfstep(j, p):
            pltpu.make_async_copy(
                u_s.at[src_v.at[j]], rows[p], gsem[p]).wait()

            @pl.when(j >= 1)
            def _():
                pltpu.make_async_copy(
                    rows[1 - p], acc.at[dst_v.at[j - 1]], ssem[1 - p]).wait()

            pltpu.async_copy(rows[p], acc.at[dst_v.at[j]], ssem[p], add=True)

            @pl.when(j + 1 < K)
            def _():
                pltpu.async_copy(
                    u_s.at[src_v.at[j + 1]], rows[1 - p], gsem[1 - p])

        def step(t, carry):
            halfstep(2 * t, 0)
            halfstep(2 * t + 1, 1)
            return carry

        lax.fori_loop(0, K // 2, step, 0)
        pltpu.make_async_copy(
            rows[1], acc.at[dst_v.at[K - 1]], ssem[1]).wait()
        plsc.subcore_barrier()
        pltpu.sync_copy(acc.at[pl.ds(base, RPT)],
                        out_hbm.at[c].at[pl.ds(base, RPT)])

    return pl.kernel(
        body,
        out_type=jax.ShapeDtypeStruct((NC, R, 64), jnp.float32),
        mesh=_sc_mesh(),
        compiler_params=pltpu.CompilerParams(use_tc_tiling_on_sc=False),
        scratch_types=[
            pltpu.VMEM_SHARED((R, 64), jnp.float32),
            pltpu.VMEM_SHARED((N, 64), jnp.float32),
            pltpu.VMEM((K, CH), jnp.int32),
            pltpu.VMEM((K, CH), jnp.int32),
            [pltpu.VMEM((CH, 64), jnp.float32) for _ in range(2)],
            [pltpu.SemaphoreType.DMA for _ in range(2)],
            [pltpu.SemaphoreType.DMA for _ in range(2)],
        ],
    )(u_tab, src3, dst3, zeros64)


_RB = 1000


def _tc_stage1a(x, W1):
    def body(x_ref, w_ref, xw_ref):
        xw_ref[...] = jnp.dot(x_ref[...], w_ref[...],
                              preferred_element_type=jnp.float32)

    grid = (N // _RB,)
    return pl.pallas_call(
        body,
        grid=grid,
        in_specs=[
            pl.BlockSpec((_RB, 128), lambda i: (i, 0)),
            pl.BlockSpec((128, 64), lambda i: (0, 0)),
        ],
        out_specs=pl.BlockSpec((_RB, 64), lambda i: (i, 0)),
        out_shape=jax.ShapeDtypeStruct((N, 64), jnp.float32),
    )(x, W1)


def _tc_stage1b(xw, degs):
    def body(xw_ref, d_ref, u_ref, dis_ref, inv_ref):
        cnt = d_ref[0] + d_ref[1] + 1.0
        dis = lax.rsqrt(cnt)
        u_ref[...] = xw_ref[...] * dis[:, 0:1]
        dis_ref[...] = dis
        inv_ref[...] = 1.0 / cnt

    grid = (N // _RB,)
    return pl.pallas_call(
        body,
        grid=grid,
        in_specs=[
            pl.BlockSpec((_RB, 64), lambda i: (i, 0)),
            pl.BlockSpec((NC, _RB, 16), lambda i: (0, i, 0)),
        ],
        out_specs=[
            pl.BlockSpec((_RB, 64), lambda i: (i, 0)),
            pl.BlockSpec((_RB, 16), lambda i: (i, 0)),
            pl.BlockSpec((_RB, 16), lambda i: (i, 0)),
        ],
        out_shape=[
            jax.ShapeDtypeStruct((N, 64), jnp.float32),
            jax.ShapeDtypeStruct((N, 16), jnp.float32),
            jax.ShapeDtypeStruct((N, 16), jnp.float32),
        ],
    )(xw, degs)


def _tc_stage3(sa, xw, dis, inv, b1):
    def body(a_ref, xw_ref, dis_ref, inv_ref, bias_ref, h_ref, v_ref):
        d = dis_ref[:, 0:1]
        h = d * (a_ref[0] + a_ref[1]) + xw_ref[...] * inv_ref[:, 0:1] \
            + bias_ref[...]
        h_ref[...] = h
        v_ref[...] = h * d

    grid = (N // _RB,)
    return pl.pallas_call(
        body,
        grid=grid,
        in_specs=[
            pl.BlockSpec((NC, _RB, 64), lambda i: (0, i, 0)),
            pl.BlockSpec((_RB, 64), lambda i: (i, 0)),
            pl.BlockSpec((_RB, 16), lambda i: (i, 0)),
            pl.BlockSpec((_RB, 16), lambda i: (i, 0)),
            pl.BlockSpec((1, 64), lambda i: (0, 0)),
        ],
        out_specs=[
            pl.BlockSpec((_RB, 64), lambda i: (i, 0)),
            pl.BlockSpec((_RB, 64), lambda i: (i, 0)),
        ],
        out_shape=[
            jax.ShapeDtypeStruct((N, 64), jnp.float32),
            jax.ShapeDtypeStruct((N, 64), jnp.float32),
        ],
    )(sa, xw, dis, inv, b1)


def _tc_stage5(sb, h, dis, inv, W2, b2, W3, b3, noise):
    def body(a_ref, h_ref, dis_ref, inv_ref, w2_ref, b2_ref,
             w3_ref, b3_ref, n_ref, z_ref):
        g = dis_ref[:, 0:1] * (a_ref[0] + a_ref[1]) \
            + h_ref[...] * inv_ref[:, 0:1]
        mean = jnp.dot(g, w2_ref[...], preferred_element_type=jnp.float32) \
            + b2_ref[...]
        logs = jnp.dot(g, w3_ref[...], preferred_element_type=jnp.float32) \
            + b3_ref[...]
        z_ref[...] = mean + n_ref[...] * jnp.exp(logs)

    grid = (N // _RB,)
    return pl.pallas_call(
        body,
        grid=grid,
        in_specs=[
            pl.BlockSpec((NC, _RB, 64), lambda i: (0, i, 0)),
            pl.BlockSpec((_RB, 64), lambda i: (i, 0)),
            pl.BlockSpec((_RB, 16), lambda i: (i, 0)),
            pl.BlockSpec((_RB, 16), lambda i: (i, 0)),
            pl.BlockSpec((64, 32), lambda i: (0, 0)),
            pl.BlockSpec((1, 32), lambda i: (0, 0)),
            pl.BlockSpec((64, 32), lambda i: (0, 0)),
            pl.BlockSpec((1, 32), lambda i: (0, 0)),
            pl.BlockSpec((_RB, 32), lambda i: (i, 0)),
        ],
        out_specs=pl.BlockSpec((_RB, 32), lambda i: (i, 0)),
        out_shape=jax.ShapeDtypeStruct((N, 32), jnp.float32),
    )(sb, h, dis, inv, W2, b2, W3, b3, noise)


_BM = 1000
_BN = 1024


def _tc_decoder(z):
    def body(zi_ref, zj_ref, o_ref):
        acc = lax.dot_general(
            zi_ref[...], zj_ref[...], (((1,), (1,)), ((), ())),
            preferred_element_type=jnp.float32)
        o_ref[...] = 1.0 / (1.0 + jnp.exp(-acc))

    grid = (N // _BM, pl.cdiv(N, _BN))
    return pl.pallas_call(
        body,
        grid=grid,
        in_specs=[
            pl.BlockSpec((_BM, 32), lambda i, j: (i, 0)),
            pl.BlockSpec((_BN, 32), lambda i, j: (j, 0)),
        ],
        out_specs=pl.BlockSpec((_BM, _BN), lambda i, j: (i, j)),
        out_shape=jax.ShapeDtypeStruct((N, N), jnp.float32),
    )(z, z)


def kernel(features, edge_index, W1, b1, W2, b2, W3, b3, noise):
    ei = edge_index.astype(jnp.int32)
    src3 = ei[0].reshape(NW, K, CH)
    dst3 = ei[1].reshape(NW, K, CH)

    const16 = jnp.concatenate(
        [jnp.ones((CH, 16), jnp.float32), jnp.zeros((IZ, 16), jnp.float32)])
    zeros64 = jnp.zeros((IZ, 64), jnp.float32)

    degs = _sc_degree(dst3, const16)
    xw = _tc_stage1a(features, W1)
    u, dis, inv = _tc_stage1b(xw, degs)
    sa = _sc_scatter64(u, src3, dst3, zeros64)
    h, v = _tc_stage3(sa, xw, dis, inv, b1.reshape(1, 64))
    sb = _sc_scatter64(v, src3, dst3, zeros64)
    z = _tc_stage5(sb, h, dis, inv,
                   W2, b2.reshape(1, 32), W3, b3.reshape(1, 32), noise)
    return _tc_decoder(z)


# speedup vs baseline: 1.0249x; 1.0249x over previous
import jax
import jax.numpy as jnp
from jax import lax
from jax.experimental import pallas as pl
from jax.experimental.pallas import tpu as pltpu
from jax.experimental.pallas import tpu_sc as plsc

N = 10000
E = 320000
NC = 2
NS = 16
NW = NC * NS
CH = 125
K = 80
R = 10240
RPT = R // NS
IZ = 128


def _wid():
    return lax.axis_index("c") * NS + lax.axis_index("s")


def _sc_mesh():
    return plsc.VectorSubcoreMesh(core_axis_name="c", subcore_axis_name="s")


def _sc_degree(dst3, const16):
    def body(dst_hbm, const_hbm, out_hbm, acc, dst_v, ones_v):
        c = lax.axis_index("c")
        s = lax.axis_index("s")
        pltpu.sync_copy(dst_hbm.at[_wid()], dst_v)
        pltpu.sync_copy(const_hbm.at[pl.ds(0, CH)], ones_v)
        base = s * RPT
        for t in range(RPT // IZ):
            pltpu.sync_copy(const_hbm.at[pl.ds(CH, IZ)],
                            acc.at[pl.ds(base + t * IZ, IZ)])
        plsc.subcore_barrier()

        def step(j, carry):
            pltpu.sync_copy(ones_v, acc.at[dst_v.at[j]], add=True)
            return carry

        lax.fori_loop(0, K, step, 0)
        plsc.subcore_barrier()
        pltpu.sync_copy(acc.at[pl.ds(base, RPT)],
                        out_hbm.at[c].at[pl.ds(base, RPT)])

    return pl.kernel(
        body,
        out_type=jax.ShapeDtypeStruct((NC, R, 16), jnp.float32),
        mesh=_sc_mesh(),
        compiler_params=pltpu.CompilerParams(use_tc_tiling_on_sc=False),
        scratch_types=[
            pltpu.VMEM_SHARED((R, 16), jnp.float32),
            pltpu.VMEM((K, CH), jnp.int32),
            pltpu.VMEM((CH, 16), jnp.float32),
        ],
    )(dst3, const16)


def _sc_scatter64(u_tab, src3, dst3, zeros64):
    def body(u_hbm, src_hbm, dst_hbm, z_hbm, out_hbm,
             acc, u_s, src_v, dst_v, rows, gsem, ssem):
        c = lax.axis_index("c")
        s = lax.axis_index("s")
        w = _wid()
        pltpu.sync_copy(src_hbm.at[w], src_v)
        pltpu.sync_copy(dst_hbm.at[w], dst_v)
        pltpu.sync_copy(u_hbm.at[pl.ds(s * (N // NS), N // NS)],
                        u_s.at[pl.ds(s * (N // NS), N // NS)])
        base = s * RPT
        for t in range(RPT // IZ):
            pltpu.sync_copy(z_hbm, acc.at[pl.ds(base + t * IZ, IZ)])
        plsc.subcore_barrier()

        pltpu.async_copy(u_s.at[src_v.at[0]], rows[0], gsem[0])

        def halfstep(j, p):
            pltpu.make_async_copy(
                u_s.at[src_v.at[j]], rows[p], gsem[p]).wait()

            @pl.when(j >= 1)
            def _():
                pltpu.make_async_copy(
                    rows[1 - p], acc.at[dst_v.at[j - 1]], ssem[1 - p]).wait()

            pltpu.async_copy(rows[p], acc.at[dst_v.at[j]], ssem[p], add=True)

            @pl.when(j + 1 < K)
            def _():
                pltpu.async_copy(
                    u_s.at[src_v.at[j + 1]], rows[1 - p], gsem[1 - p])

        def step(t, carry):
            halfstep(2 * t, 0)
            halfstep(2 * t + 1, 1)
            return carry

        lax.fori_loop(0, K // 2, step, 0)
        pltpu.make_async_copy(
            rows[1], acc.at[dst_v.at[K - 1]], ssem[1]).wait()
        plsc.subcore_barrier()
        pltpu.sync_copy(acc.at[pl.ds(base, RPT)],
                        out_hbm.at[c].at[pl.ds(base, RPT)])

    return pl.kernel(
        body,
        out_type=jax.ShapeDtypeStruct((NC, R, 64), jnp.float32),
        mesh=_sc_mesh(),
        compiler_params=pltpu.CompilerParams(use_tc_tiling_on_sc=False),
        scratch_types=[
            pltpu.VMEM_SHARED((R, 64), jnp.float32),
            pltpu.VMEM_SHARED((N, 64), jnp.float32),
            pltpu.VMEM((K, CH), jnp.int32),
            pltpu.VMEM((K, CH), jnp.int32),
            [pltpu.VMEM((CH, 64), jnp.float32) for _ in range(2)],
            [pltpu.SemaphoreType.DMA for _ in range(2)],
            [pltpu.SemaphoreType.DMA for _ in range(2)],
        ],
    )(u_tab, src3, dst3, zeros64)


_RB = 1000


def _tc_stage1(x, W1, degs):
    def body(x_ref, w_ref, d_ref, xw_ref, u_ref, dis_ref, inv_ref):
        xw = jnp.dot(x_ref[...], w_ref[...], preferred_element_type=jnp.float32)
        cnt = d_ref[0] + d_ref[1] + 1.0
        dis = lax.rsqrt(cnt)
        inv = 1.0 / cnt
        xw_ref[...] = xw
        u_ref[...] = xw * dis[:, 0:1]
        dis_ref[...] = dis
        inv_ref[...] = inv

    grid = (N // _RB,)
    return pl.pallas_call(
        body,
        grid=grid,
        in_specs=[
            pl.BlockSpec((_RB, 128), lambda i: (i, 0)),
            pl.BlockSpec((128, 64), lambda i: (0, 0)),
            pl.BlockSpec((NC, _RB, 16), lambda i: (0, i, 0)),
        ],
        out_specs=[
            pl.BlockSpec((_RB, 64), lambda i: (i, 0)),
            pl.BlockSpec((_RB, 64), lambda i: (i, 0)),
            pl.BlockSpec((_RB, 16), lambda i: (i, 0)),
            pl.BlockSpec((_RB, 16), lambda i: (i, 0)),
        ],
        out_shape=[
            jax.ShapeDtypeStruct((N, 64), jnp.float32),
            jax.ShapeDtypeStruct((N, 64), jnp.float32),
            jax.ShapeDtypeStruct((N, 16), jnp.float32),
            jax.ShapeDtypeStruct((N, 16), jnp.float32),
        ],
    )(x, W1, degs)


def _tc_stage3(sa, xw, dis, inv, b1):
    def body(a_ref, xw_ref, dis_ref, inv_ref, bias_ref, h_ref, v_ref):
        d = dis_ref[:, 0:1]
        h = d * (a_ref[0] + a_ref[1]) + xw_ref[...] * inv_ref[:, 0:1] \
            + bias_ref[...]
        h_ref[...] = h
        v_ref[...] = h * d

    grid = (N // _RB,)
    return pl.pallas_call(
        body,
        grid=grid,
        in_specs=[
            pl.BlockSpec((NC, _RB, 64), lambda i: (0, i, 0)),
            pl.BlockSpec((_RB, 64), lambda i: (i, 0)),
            pl.BlockSpec((_RB, 16), lambda i: (i, 0)),
            pl.BlockSpec((_RB, 16), lambda i: (i, 0)),
            pl.BlockSpec((1, 64), lambda i: (0, 0)),
        ],
        out_specs=[
            pl.BlockSpec((_RB, 64), lambda i: (i, 0)),
            pl.BlockSpec((_RB, 64), lambda i: (i, 0)),
        ],
        out_shape=[
            jax.ShapeDtypeStruct((N, 64), jnp.float32),
            jax.ShapeDtypeStruct((N, 64), jnp.float32),
        ],
    )(sa, xw, dis, inv, b1)


def _tc_stage5(sb, h, dis, inv, W2, b2, W3, b3, noise):
    def body(a_ref, h_ref, dis_ref, inv_ref, w2_ref, b2_ref,
             w3_ref, b3_ref, n_ref, z_ref):
        g = dis_ref[:, 0:1] * (a_ref[0] + a_ref[1]) \
            + h_ref[...] * inv_ref[:, 0:1]
        mean = jnp.dot(g, w2_ref[...], preferred_element_type=jnp.float32) \
            + b2_ref[...]
        logs = jnp.dot(g, w3_ref[...], preferred_element_type=jnp.float32) \
            + b3_ref[...]
        z_ref[...] = mean + n_ref[...] * jnp.exp(logs)

    grid = (N // _RB,)
    return pl.pallas_call(
        body,
        grid=grid,
        in_specs=[
            pl.BlockSpec((NC, _RB, 64), lambda i: (0, i, 0)),
            pl.BlockSpec((_RB, 64), lambda i: (i, 0)),
            pl.BlockSpec((_RB, 16), lambda i: (i, 0)),
            pl.BlockSpec((_RB, 16), lambda i: (i, 0)),
            pl.BlockSpec((64, 32), lambda i: (0, 0)),
            pl.BlockSpec((1, 32), lambda i: (0, 0)),
            pl.BlockSpec((64, 32), lambda i: (0, 0)),
            pl.BlockSpec((1, 32), lambda i: (0, 0)),
            pl.BlockSpec((_RB, 32), lambda i: (i, 0)),
        ],
        out_specs=pl.BlockSpec((_RB, 32), lambda i: (i, 0)),
        out_shape=jax.ShapeDtypeStruct((N, 32), jnp.float32),
    )(sb, h, dis, inv, W2, b2, W3, b3, noise)


_BM = 1000
_BN = 1024


def _tc_decoder(z):
    def body(zi_ref, zj_ref, o_ref):
        acc = lax.dot_general(
            zi_ref[...], zj_ref[...], (((1,), (1,)), ((), ())),
            preferred_element_type=jnp.float32)
        o_ref[...] = 1.0 / (1.0 + jnp.exp(-acc))

    grid = (N // _BM, pl.cdiv(N, _BN))
    return pl.pallas_call(
        body,
        grid=grid,
        in_specs=[
            pl.BlockSpec((_BM, 32), lambda i, j: (i, 0)),
            pl.BlockSpec((_BN, 32), lambda i, j: (j, 0)),
        ],
        out_specs=pl.BlockSpec((_BM, _BN), lambda i, j: (i, j)),
        out_shape=jax.ShapeDtypeStruct((N, N), jnp.float32),
    )(z, z)


def kernel(features, edge_index, W1, b1, W2, b2, W3, b3, noise):
    ei = edge_index.astype(jnp.int32)
    src3 = ei[0].reshape(NW, K, CH)
    dst3 = ei[1].reshape(NW, K, CH)

    const16 = jnp.concatenate(
        [jnp.ones((CH, 16), jnp.float32), jnp.zeros((IZ, 16), jnp.float32)])
    zeros64 = jnp.zeros((IZ, 64), jnp.float32)

    degs = _sc_degree(dst3, const16)
    xw, u, dis, inv = _tc_stage1(features, W1, degs)
    sa = _sc_scatter64(u, src3, dst3, zeros64)
    h, v = _tc_stage3(sa, xw, dis, inv, b1.reshape(1, 64))
    sb = _sc_scatter64(v, src3, dst3, zeros64)
    z = _tc_stage5(sb, h, dis, inv,
                   W2, b2.reshape(1, 32), W3, b3.reshape(1, 32), noise)
    return _tc_decoder(z)
